# baseline (device time: 282747 ns/iter reference)
import functools

import jax
import jax.numpy as jnp
from jax import lax
from jax.experimental import pallas as pl
from jax.experimental.pallas import tpu as pltpu

N_DEV = 4
SQ = 2048
SKV = 2048
HQ = 8
DH = 128
DM = 1024
BLK = 64
QC = 512
N_CHUNK = SQ // QC
SCALE = 0.08838834764831843


def kernel(x, Wq, K_ext, V_ext, Wo):
    my = lax.axis_index("i")
    xb = x[0].astype(jnp.bfloat16)
    wq = (Wq * SCALE).astype(jnp.bfloat16)
    wo = Wo.astype(jnp.bfloat16)
    kb = lax.dynamic_index_in_dim(K_ext, my, 0, keepdims=False)
    vb = lax.dynamic_index_in_dim(V_ext, my, 0, keepdims=False)
    kb = jnp.transpose(kb, (1, 0, 2)).astype(jnp.bfloat16)
    vb = jnp.transpose(vb, (1, 0, 2)).astype(jnp.bfloat16)

    def body(x_ref, k_hbm, v_hbm, wq_ref, wo_ref, out_ref,
             comm_ref, kg_ref, vg_ref,
             send_sems, recv_sems, kv_sems):
        my_pos = lax.axis_index("i")
        left = lax.rem(my_pos + N_DEV - 1, N_DEV)
        right = lax.rem(my_pos + 1, N_DEV)

        barrier = pltpu.get_barrier_semaphore()
        for nbr in (left, right):
            pl.semaphore_signal(barrier, inc=1, device_id=(nbr,),
                                device_id_type=pl.DeviceIdType.MESH)
        pl.semaphore_wait(barrier, 2)

        comm_ref[0, 0, :, :] = wq_ref[:, :]
        comm_ref[0, 1, :, :] = wo_ref[:, :]
        out_ref[...] = jnp.zeros_like(out_ref)

        qb_i = lax.broadcasted_iota(jnp.int32, (QC, QC), 0) // BLK
        kb_i = lax.broadcasted_iota(jnp.int32, (QC, QC), 1) // BLK
        diag_neg = jnp.where(kb_i <= qb_i, 0.0, -1e9).astype(jnp.float32)

        def hop(h, carry):
            g = lax.rem(my_pos - h + N_DEV, N_DEV)
            nxt = jnp.minimum(h + 1, N_DEV - 1)

            rdma = pltpu.make_async_remote_copy(
                src_ref=comm_ref.at[h],
                dst_ref=comm_ref.at[nxt],
                send_sem=send_sems.at[h],
                recv_sem=recv_sems.at[nxt],
                device_id=(right,),
                device_id_type=pl.DeviceIdType.MESH,
            )

            @pl.when(h < N_DEV - 1)
            def _():
                rdma.start()

            kcp = pltpu.make_async_copy(
                k_hbm.at[pl.ds(g * HQ, HQ)], kg_ref, kv_sems.at[0])
            vcp = pltpu.make_async_copy(
                v_hbm.at[pl.ds(g * HQ, HQ)], vg_ref, kv_sems.at[1])
            kcp.start()
            vcp.start()
            kcp.wait()
            vcp.wait()

            wq_g = comm_ref[h, 0]
            wo_g = comm_ref[h, 1]

            cc = (((1,), (1,)), ((), ()))
            for c in range(N_CHUNK):
                xc = x_ref[c * QC:(c + 1) * QC, :]
                q_c = jnp.dot(
                    xc, wq_g,
                    preferred_element_type=jnp.float32).astype(jnp.bfloat16)
                ctx_parts = []
                for hd in range(HQ):
                    q_hd = q_c[:, hd * DH:(hd + 1) * DH]
                    p_d = jnp.exp(lax.dot_general(
                        q_hd, kg_ref[hd, c * QC:(c + 1) * QC], cc,
                        preferred_element_type=jnp.float32) + diag_neg)
                    denom = jnp.sum(p_d, axis=1, keepdims=True)
                    ctx = jnp.dot(
                        p_d.astype(jnp.bfloat16),
                        vg_ref[hd, c * QC:(c + 1) * QC],
                        preferred_element_type=jnp.float32)
                    if c > 0:
                        p_v = jnp.exp(lax.dot_general(
                            q_hd, kg_ref[hd, :c * QC], cc,
                            preferred_element_type=jnp.float32))
                        denom = denom + jnp.sum(p_v, axis=1, keepdims=True)
                        ctx = ctx + jnp.dot(
                            p_v.astype(jnp.bfloat16), vg_ref[hd, :c * QC],
                            preferred_element_type=jnp.float32)
                    ctx_parts.append((ctx / denom).astype(jnp.bfloat16))
                ctx_c = jnp.concatenate(ctx_parts, axis=1)
                out_ref[c * QC:(c + 1) * QC, :] += jnp.dot(
                    ctx_c, wo_g, preferred_element_type=jnp.float32)

            @pl.when(h < N_DEV - 1)
            def _():
                rdma.wait()

            return carry

        lax.fori_loop(0, N_DEV, hop, 0)

        @functools.partial(pl.run_scoped,
                           sem2=pltpu.SemaphoreType.REGULAR)
        def _(sem2):
            for nbr in (left, right):
                pl.semaphore_signal(sem2, inc=1, device_id=(nbr,),
                                    device_id_type=pl.DeviceIdType.MESH)
            pl.semaphore_wait(sem2, 2)

    out = pl.pallas_call(
        body,
        out_shape=jax.ShapeDtypeStruct((SQ, DM), jnp.float32),
        in_specs=[
            pl.BlockSpec(memory_space=pltpu.VMEM),
            pl.BlockSpec(memory_space=pl.ANY),
            pl.BlockSpec(memory_space=pl.ANY),
            pl.BlockSpec(memory_space=pltpu.VMEM),
            pl.BlockSpec(memory_space=pltpu.VMEM),
        ],
        out_specs=pl.BlockSpec(memory_space=pltpu.VMEM),
        scratch_shapes=[
            pltpu.VMEM((N_DEV, 2, DM, DM), jnp.bfloat16),
            pltpu.VMEM((HQ, SKV, DH), jnp.bfloat16),
            pltpu.VMEM((HQ, SKV, DH), jnp.bfloat16),
            pltpu.SemaphoreType.DMA((N_DEV,)),
            pltpu.SemaphoreType.DMA((N_DEV,)),
            pltpu.SemaphoreType.DMA((2,)),
        ],
        compiler_params=pltpu.CompilerParams(
            collective_id=0,
            vmem_limit_bytes=56 * 1024 * 1024,
        ),
    )(xb, kb, vb, wq, wo)
    return out[None]


# device time: 207549 ns/iter; 1.3623x vs baseline; 1.3623x over previous
import functools

import jax
import jax.numpy as jnp
from jax import lax
from jax.experimental import pallas as pl
from jax.experimental.pallas import tpu as pltpu

N_DEV = 4
SQ = 2048
SKV = 2048
HQ = 8
DH = 128
DM = 1024
BLK = 64
QC = 512
N_CHUNK = SQ // QC
SCALE = 0.08838834764831843


def kernel(x, Wq, K_ext, V_ext, Wo):
    xb = x[0].astype(jnp.bfloat16)
    wq = (Wq * SCALE).astype(jnp.bfloat16)
    wo = Wo.astype(jnp.bfloat16)

    def body(x_ref, k_hbm, v_hbm, wq_ref, wo_ref, out_ref,
             comm_ref, kg_ref, vg_ref, kst_ref, vst_ref,
             send_sems, recv_sems, kv_sems):
        my_pos = lax.axis_index("i")
        left = lax.rem(my_pos + N_DEV - 1, N_DEV)
        right = lax.rem(my_pos + 1, N_DEV)

        barrier = pltpu.get_barrier_semaphore()
        for nbr in (left, right):
            pl.semaphore_signal(barrier, inc=1, device_id=(nbr,),
                                device_id_type=pl.DeviceIdType.MESH)
        pl.semaphore_wait(barrier, 2)

        comm_ref[0, 0, :, :] = wq_ref[:, :]
        comm_ref[0, 1, :, :] = wo_ref[:, :]
        out_ref[...] = jnp.zeros_like(out_ref)

        qb_i = lax.broadcasted_iota(jnp.int32, (QC, QC), 0) // BLK
        kb_i = lax.broadcasted_iota(jnp.int32, (QC, QC), 1) // BLK
        diag_neg = jnp.where(kb_i <= qb_i, 0.0, -1e9).astype(jnp.float32)

        def hop(h, carry):
            g = lax.rem(my_pos - h + N_DEV, N_DEV)
            nxt = jnp.minimum(h + 1, N_DEV - 1)

            rdma = pltpu.make_async_remote_copy(
                src_ref=comm_ref.at[h],
                dst_ref=comm_ref.at[nxt],
                send_sem=send_sems.at[h],
                recv_sem=recv_sems.at[nxt],
                device_id=(right,),
                device_id_type=pl.DeviceIdType.MESH,
            )

            @pl.when(h < N_DEV - 1)
            def _():
                rdma.start()

            base = g * HQ

            def kv_dma(hd, slot):
                kcp = pltpu.make_async_copy(
                    k_hbm.at[my_pos, :, base + hd, :],
                    kst_ref.at[slot], kv_sems.at[0, slot])
                vcp = pltpu.make_async_copy(
                    v_hbm.at[my_pos, :, base + hd, :],
                    vst_ref.at[slot], kv_sems.at[1, slot])
                return kcp, vcp

            k0, v0 = kv_dma(0, 0)
            k0.start()
            v0.start()
            for hd in range(HQ):
                slot = hd % 2
                if hd + 1 < HQ:
                    kn, vn = kv_dma(hd + 1, 1 - slot)
                    kn.start()
                    vn.start()
                kc, vc = kv_dma(hd, slot)
                kc.wait()
                vc.wait()
                kg_ref[hd] = kst_ref[slot].astype(jnp.bfloat16)
                vg_ref[hd] = vst_ref[slot].astype(jnp.bfloat16)

            wq_g = comm_ref[h, 0]
            wo_g = comm_ref[h, 1]

            cc = (((1,), (1,)), ((), ()))
            for c in range(N_CHUNK):
                xc = x_ref[c * QC:(c + 1) * QC, :]
                q_c = jnp.dot(
                    xc, wq_g,
                    preferred_element_type=jnp.float32).astype(jnp.bfloat16)
                ctx_parts = []
                for hd in range(HQ):
                    q_hd = q_c[:, hd * DH:(hd + 1) * DH]
                    p_d = jnp.exp(lax.dot_general(
                        q_hd, kg_ref[hd, c * QC:(c + 1) * QC], cc,
                        preferred_element_type=jnp.float32) + diag_neg)
                    denom = jnp.sum(p_d, axis=1, keepdims=True)
                    ctx = jnp.dot(
                        p_d.astype(jnp.bfloat16),
                        vg_ref[hd, c * QC:(c + 1) * QC],
                        preferred_element_type=jnp.float32)
                    if c > 0:
                        p_v = jnp.exp(lax.dot_general(
                            q_hd, kg_ref[hd, :c * QC], cc,
                            preferred_element_type=jnp.float32))
                        denom = denom + jnp.sum(p_v, axis=1, keepdims=True)
                        ctx = ctx + jnp.dot(
                            p_v.astype(jnp.bfloat16), vg_ref[hd, :c * QC],
                            preferred_element_type=jnp.float32)
                    ctx_parts.append((ctx / denom).astype(jnp.bfloat16))
                ctx_c = jnp.concatenate(ctx_parts, axis=1)
                out_ref[c * QC:(c + 1) * QC, :] += jnp.dot(
                    ctx_c, wo_g, preferred_element_type=jnp.float32)

            @pl.when(h < N_DEV - 1)
            def _():
                rdma.wait()

            return carry

        lax.fori_loop(0, N_DEV, hop, 0)

        @functools.partial(pl.run_scoped,
                           sem2=pltpu.SemaphoreType.REGULAR)
        def _(sem2):
            for nbr in (left, right):
                pl.semaphore_signal(sem2, inc=1, device_id=(nbr,),
                                    device_id_type=pl.DeviceIdType.MESH)
            pl.semaphore_wait(sem2, 2)

    out = pl.pallas_call(
        body,
        out_shape=jax.ShapeDtypeStruct((SQ, DM), jnp.float32),
        in_specs=[
            pl.BlockSpec(memory_space=pltpu.VMEM),
            pl.BlockSpec(memory_space=pl.ANY),
            pl.BlockSpec(memory_space=pl.ANY),
            pl.BlockSpec(memory_space=pltpu.VMEM),
            pl.BlockSpec(memory_space=pltpu.VMEM),
        ],
        out_specs=pl.BlockSpec(memory_space=pltpu.VMEM),
        scratch_shapes=[
            pltpu.VMEM((N_DEV, 2, DM, DM), jnp.bfloat16),
            pltpu.VMEM((HQ, SKV, DH), jnp.bfloat16),
            pltpu.VMEM((HQ, SKV, DH), jnp.bfloat16),
            pltpu.VMEM((2, SKV, DH), jnp.float32),
            pltpu.VMEM((2, SKV, DH), jnp.float32),
            pltpu.SemaphoreType.DMA((N_DEV,)),
            pltpu.SemaphoreType.DMA((N_DEV,)),
            pltpu.SemaphoreType.DMA((2, 2)),
        ],
        compiler_params=pltpu.CompilerParams(
            collective_id=0,
            vmem_limit_bytes=60 * 1024 * 1024,
        ),
    )(xb, K_ext, V_ext, wq, wo)
    return out[None]
